# baseline (speedup 1.0000x reference)
import functools

import jax
import jax.numpy as jnp
from jax.experimental import pallas as pl
from jax.experimental.pallas import tpu as pltpu

_FEATS = 32768
_ROWS = 128
_BR = 64
_EPS = 1e-5
_BASE_SPARSITY = 0.1


def _tree_sum(v):
    f = v.shape[-1]
    while f > 128:
        f //= 2
        v = v[:, :f] + v[:, f:]
    return jnp.sum(v, axis=-1, keepdims=True)


def _asl_body(vs_ref, x_ref, g_ref, b_ref, o_ref, hi_ref, lo_ref, lom_ref,
              k_ref):
    @pl.when(pl.program_id(0) == 0)
    def _():
        avg = jnp.clip(_tree_sum(vs_ref[...])[0, 0] * (1.0 / _FEATS),
                       0.1, 2.0)
        sp = jnp.clip(_BASE_SPARSITY * (1.0 + 0.5 * (avg - 1.0)), 0.05, 0.25)
        k_ref[0] = jnp.maximum(1, (sp * _FEATS).astype(jnp.int32))

    k = k_ref[0]

    x = x_ref[...]
    inv_f = 1.0 / _FEATS
    mean = _tree_sum(x) * inv_f
    msq = _tree_sum(x * x) * inv_f
    var = msq - mean * mean
    xn = (x - mean) * jax.lax.rsqrt(var + _EPS) * g_ref[...] + b_ref[...]
    o_ref[...] = xn

    i32 = jax.lax.bitcast_convert_type(xn, jnp.int32)
    s = i32 ^ ((i32 >> 31) & jnp.int32(0x7FFFFFFF))
    hi_ref[...] = (s >> 16).astype(jnp.int16)
    lo_ref[...] = ((s & 0xFFFF) - 32768).astype(jnp.int16)

    nrow = x.shape[0]
    nacc = 4
    chunk = 256
    chunks = _FEATS // chunk

    def _count16(ref, cand_u):
        cand_s = (cand_u - 32768).astype(jnp.int16)
        accs = [jnp.zeros((nrow, chunk), jnp.int32) for _ in range(nacc)]
        for c in range(chunks):
            blk = ref[:, c * chunk:(c + 1) * chunk]
            accs[c % nacc] = accs[c % nacc] + (blk >= cand_s).astype(jnp.int32)
        acc = (accs[0] + accs[1]) + (accs[2] + accs[3])
        return jnp.sum(acc, axis=-1, keepdims=True)

    def bit_a(idx, t_u):
        cand_u = t_u | (jnp.int32(1) << (15 - idx))
        cnt = _count16(hi_ref, cand_u)
        return jnp.where(cnt >= k, cand_u, t_u)

    t16_u = jax.lax.fori_loop(0, 16, bit_a, jnp.zeros((nrow, 1), jnp.int32))
    t16_s = (t16_u - 32768).astype(jnp.int16)

    n_hi = _count16(hi_ref, t16_u + 1)
    rem = k - n_hi

    lom_ref[...] = jnp.where(hi_ref[...] == t16_s, lo_ref[...],
                             jnp.int16(-32768))

    def bit_b(idx, t_u):
        cand_u = t_u | (jnp.int32(1) << (15 - idx))
        cnt = _count16(lom_ref, cand_u)
        return jnp.where(cnt >= rem, cand_u, t_u)

    tlo_u = jax.lax.fori_loop(0, 16, bit_b, jnp.zeros((nrow, 1), jnp.int32))
    tlo_s = (tlo_u - 32768).astype(jnp.int16)

    hi = hi_ref[...]
    keep = (hi > t16_s) | ((hi == t16_s) & (lo_ref[...] >= tlo_s))
    o_ref[...] = jnp.where(keep, o_ref[...], 0.0)


@jax.jit
def kernel(x, variance_signal, gamma, beta):
    vs2 = variance_signal.reshape(1, _FEATS)
    g2 = gamma.reshape(1, _FEATS)
    b2 = beta.reshape(1, _FEATS)
    grid = (_ROWS // _BR,)
    return pl.pallas_call(
        _asl_body,
        grid=grid,
        in_specs=[
            pl.BlockSpec((1, _FEATS), lambda i: (0, 0)),
            pl.BlockSpec((_BR, _FEATS), lambda i: (i, 0)),
            pl.BlockSpec((1, _FEATS), lambda i: (0, 0)),
            pl.BlockSpec((1, _FEATS), lambda i: (0, 0)),
        ],
        out_specs=pl.BlockSpec((_BR, _FEATS), lambda i: (i, 0)),
        out_shape=jax.ShapeDtypeStruct((_ROWS, _FEATS), jnp.float32),
        scratch_shapes=[
            pltpu.VMEM((_BR, _FEATS), jnp.int16),
            pltpu.VMEM((_BR, _FEATS), jnp.int16),
            pltpu.VMEM((_BR, _FEATS), jnp.int16),
            pltpu.SMEM((1,), jnp.int32),
        ],
    )(vs2, x, g2, b2)


# speedup vs baseline: 1.5405x; 1.5405x over previous
import functools

import jax
import jax.numpy as jnp
from jax.experimental import pallas as pl
from jax.experimental.pallas import tpu as pltpu

_FEATS = 32768
_ROWS = 128
_BR = 64
_EPS = 1e-5
_BASE_SPARSITY = 0.1


def _tree_sum(v):
    f = v.shape[-1]
    while f > 128:
        f //= 2
        v = v[:, :f] + v[:, f:]
    return jnp.sum(v, axis=-1, keepdims=True)


def _asl_body(vs_ref, x_ref, g_ref, b_ref, o_ref, key_ref, k_ref):
    @pl.when(pl.program_id(0) == 0)
    def _():
        avg = jnp.clip(_tree_sum(vs_ref[...])[0, 0] * (1.0 / _FEATS),
                       0.1, 2.0)
        sp = jnp.clip(_BASE_SPARSITY * (1.0 + 0.5 * (avg - 1.0)), 0.05, 0.25)
        k_ref[0] = jnp.maximum(1, (sp * _FEATS).astype(jnp.int32))

    k = k_ref[0]

    x = x_ref[...]
    inv_f = 1.0 / _FEATS
    mean = _tree_sum(x) * inv_f
    msq = _tree_sum(x * x) * inv_f
    var = msq - mean * mean
    xn = (x - mean) * jax.lax.rsqrt(var + _EPS) * g_ref[...] + b_ref[...]
    o_ref[...] = xn

    i32 = jax.lax.bitcast_convert_type(xn, jnp.int32)
    s = i32 ^ ((i32 >> 31) & jnp.int32(0x7FFFFFFF))
    key_ref[...] = s

    nacc = 4
    chunks = _FEATS // 128

    def bit_step(idx, t):
        b = 31 - idx
        cand = t ^ (jnp.int32(1) << b)
        accs = [jnp.zeros((x.shape[0], 128), jnp.int32) for _ in range(nacc)]
        for c in range(chunks):
            blk = key_ref[:, c * 128:(c + 1) * 128]
            accs[c % nacc] = accs[c % nacc] + (blk >= cand).astype(jnp.int32)
        acc = (accs[0] + accs[1]) + (accs[2] + accs[3])
        cnt = jnp.sum(acc, axis=-1, keepdims=True)
        return jnp.where(cnt >= k, cand, t)

    t0 = jnp.full((x.shape[0], 1), jnp.int32(-(2 ** 31)))
    t = jax.lax.fori_loop(0, 32, bit_step, t0)

    o_ref[...] = jnp.where(key_ref[...] >= t, o_ref[...], 0.0)


@jax.jit
def kernel(x, variance_signal, gamma, beta):
    vs2 = variance_signal.reshape(1, _FEATS)
    g2 = gamma.reshape(1, _FEATS)
    b2 = beta.reshape(1, _FEATS)
    grid = (_ROWS // _BR,)
    return pl.pallas_call(
        _asl_body,
        grid=grid,
        in_specs=[
            pl.BlockSpec((1, _FEATS), lambda i: (0, 0)),
            pl.BlockSpec((_BR, _FEATS), lambda i: (i, 0)),
            pl.BlockSpec((1, _FEATS), lambda i: (0, 0)),
            pl.BlockSpec((1, _FEATS), lambda i: (0, 0)),
        ],
        out_specs=pl.BlockSpec((_BR, _FEATS), lambda i: (i, 0)),
        out_shape=jax.ShapeDtypeStruct((_ROWS, _FEATS), jnp.float32),
        scratch_shapes=[
            pltpu.VMEM((_BR, _FEATS), jnp.int32),
            pltpu.SMEM((1,), jnp.int32),
        ],
    )(vs2, x, g2, b2)
